# Initial kernel scaffold; baseline (speedup 1.0000x reference)
import jax
import jax.numpy as jnp
from jax.experimental import pallas as pl


def kernel(x, edge_index, W1, b1, W2, b2):
    raise NotImplementedError("write your pallas kernel here")


# speedup vs baseline: 15.7289x; 15.7289x over previous
import functools

import jax
import jax.numpy as jnp
from jax import lax
from jax.experimental import pallas as pl
from jax.experimental.pallas import tpu as pltpu
from jax.experimental.pallas import tpu_sc as plsc

NC = 2
NS = 16
CH = 128


def _mesh():
    return plsc.VectorSubcoreMesh(core_axis_name="c", subcore_axis_name="s")


def _sc_degree(dst_pad, rows, rpw, nch, epw):
    @functools.partial(
        pl.kernel,
        out_type=jax.ShapeDtypeStruct((NC * rows,), jnp.float32),
        mesh=_mesh(),
        scratch_types=[
            pltpu.VMEM((CH,), jnp.int32),
            pltpu.VMEM((CH,), jnp.float32),
            pltpu.VMEM_SHARED((rows,), jnp.float32),
        ],
        name="sc_gcn_degree",
    )
    def deg_k(dst_hbm, out_hbm, didx, vals, acc):
        cid = lax.axis_index("c")
        sid = lax.axis_index("s")
        for k in range(CH // 16):
            vals[pl.ds(k * 16, 16)] = jnp.zeros((16,), jnp.float32)
        for r in range(rpw // CH):
            pltpu.sync_copy(vals, acc.at[pl.ds(sid * rpw + r * CH, CH)])
        plsc.subcore_barrier()
        for k in range(CH // 16):
            vals[pl.ds(k * 16, 16)] = jnp.ones((16,), jnp.float32)
        base = (cid * NS + sid) * epw

        def body(j, carry):
            pltpu.sync_copy(dst_hbm.at[pl.ds(base + j * CH, CH)], didx)
            pltpu.sync_copy(vals, acc.at[didx], add=True)
            return carry

        lax.fori_loop(0, nch, body, 0)
        plsc.subcore_barrier()
        pltpu.sync_copy(acc.at[pl.ds(sid * rpw, rpw)],
                        out_hbm.at[pl.ds(cid * rows + sid * rpw, rpw)])

    return deg_k(dst_pad)


def _sc_prop_wide(table, src_pad, dst_pad, rows, rpw, nch, epw, d):
    @functools.partial(
        pl.kernel,
        out_type=jax.ShapeDtypeStruct((NC * rows, d), jnp.float32),
        mesh=_mesh(),
        scratch_types=[
            pltpu.VMEM((CH,), jnp.int32),
            pltpu.VMEM((CH,), jnp.int32),
            pltpu.VMEM((CH, d), jnp.float32),
            pltpu.VMEM_SHARED((rows, d), jnp.float32),
            pltpu.SemaphoreType.DMA,
        ],
        name="sc_gcn_prop128",
    )
    def prop_k(tab_hbm, src_hbm, dst_hbm, out_hbm, sidx, didx, gbuf, acc, sem):
        cid = lax.axis_index("c")
        sid = lax.axis_index("s")

        def zbody(r, carry):
            for k in range(d // 16):
                gbuf[r, pl.ds(k * 16, 16)] = jnp.zeros((16,), jnp.float32)
            return carry

        lax.fori_loop(0, CH, zbody, 0)
        for r in range(rpw // CH):
            pltpu.sync_copy(gbuf, acc.at[pl.ds(sid * rpw + r * CH, CH)])
        plsc.subcore_barrier()
        base = (cid * NS + sid) * epw

        def body(j, carry):
            off = base + j * CH
            pltpu.sync_copy(src_hbm.at[pl.ds(off, CH)], sidx)
            pltpu.sync_copy(dst_hbm.at[pl.ds(off, CH)], didx)
            pltpu.async_copy(tab_hbm.at[sidx], gbuf, sem).wait()
            pltpu.sync_copy(gbuf, acc.at[didx], add=True)
            return carry

        lax.fori_loop(0, nch, body, 0)
        plsc.subcore_barrier()
        pltpu.sync_copy(acc.at[pl.ds(sid * rpw, rpw)],
                        out_hbm.at[pl.ds(cid * rows + sid * rpw, rpw)])

    return prop_k(table, src_pad, dst_pad)


def _sc_prop_scalar(vec, src_pad, dst_pad, rows, rpw, nch, epw):
    @functools.partial(
        pl.kernel,
        out_type=jax.ShapeDtypeStruct((NC * rows,), jnp.float32),
        mesh=_mesh(),
        scratch_types=[
            pltpu.VMEM((CH,), jnp.int32),
            pltpu.VMEM((CH,), jnp.int32),
            pltpu.VMEM((CH,), jnp.float32),
            pltpu.VMEM_SHARED((rows,), jnp.float32),
            pltpu.SemaphoreType.DMA,
        ],
        name="sc_gcn_prop1",
    )
    def prop1_k(vec_hbm, src_hbm, dst_hbm, out_hbm, sidx, didx, vals, acc, sem):
        cid = lax.axis_index("c")
        sid = lax.axis_index("s")
        for k in range(CH // 16):
            vals[pl.ds(k * 16, 16)] = jnp.zeros((16,), jnp.float32)
        for r in range(rpw // CH):
            pltpu.sync_copy(vals, acc.at[pl.ds(sid * rpw + r * CH, CH)])
        plsc.subcore_barrier()
        base = (cid * NS + sid) * epw

        def body(j, carry):
            off = base + j * CH
            pltpu.sync_copy(src_hbm.at[pl.ds(off, CH)], sidx)
            pltpu.sync_copy(dst_hbm.at[pl.ds(off, CH)], didx)
            pltpu.async_copy(vec_hbm.at[sidx], vals, sem).wait()
            pltpu.sync_copy(vals, acc.at[didx], add=True)
            return carry

        lax.fori_loop(0, nch, body, 0)
        plsc.subcore_barrier()
        pltpu.sync_copy(acc.at[pl.ds(sid * rpw, rpw)],
                        out_hbm.at[pl.ds(cid * rows + sid * rpw, rpw)])

    return prop1_k(vec, src_pad, dst_pad)


def _tc_scale_matmul(x, w1, d0, d1, n, d_hid):
    def body(x_ref, w_ref, d0_ref, d1_ref, h_ref, dinv_ref):
        deg = d0_ref[...] + d1_ref[...] + 1.0
        dinv = lax.rsqrt(jnp.maximum(deg, 1.0))
        h = jnp.dot(x_ref[...], w_ref[...], preferred_element_type=jnp.float32)
        h_ref[...] = h * dinv
        dinv_ref[...] = dinv

    return pl.pallas_call(
        body,
        out_shape=(jax.ShapeDtypeStruct((n, d_hid), jnp.float32),
                   jax.ShapeDtypeStruct((n, 1), jnp.float32)),
    )(x, w1, d0, d1)


def _tc_layer2_in(p0, p1, h1s, dinv, b1, w2, n):
    def body(p0_ref, p1_ref, h_ref, dinv_ref, b1_ref, w2_ref, vs_ref):
        out1 = dinv_ref[...] * (p0_ref[...] + p1_ref[...] + h_ref[...]) + b1_ref[...]
        hrelu = jnp.maximum(out1, 0.0)
        v = jnp.dot(hrelu, w2_ref[...], preferred_element_type=jnp.float32)
        vs_ref[...] = dinv_ref[...] * v

    return pl.pallas_call(
        body,
        out_shape=jax.ShapeDtypeStruct((n, 1), jnp.float32),
    )(p0, p1, h1s, dinv, b1, w2)


def _tc_finish(t0, t1, vs, dinv, b2, n):
    def body(t0_ref, t1_ref, vs_ref, dinv_ref, b2_ref, o_ref):
        z = dinv_ref[...] * (t0_ref[...] + t1_ref[...] + vs_ref[...]) + b2_ref[...]
        o_ref[...] = 1.0 / (1.0 + jnp.exp(-z))

    return pl.pallas_call(
        body,
        out_shape=jax.ShapeDtypeStruct((n, 1), jnp.float32),
    )(t0, t1, vs, dinv, b2)


def kernel(x, edge_index, W1, b1, W2, b2):
    n, d_in = x.shape
    d_hid = W1.shape[1]
    e = edge_index.shape[1]

    nw = NC * NS
    nch = -(-e // (nw * CH))
    epw = nch * CH
    e_pad = epw * nw
    rpw = -(-(n + 1) // (NS * CH)) * CH
    rows = rpw * NS

    src = edge_index[0]
    dst = edge_index[1]
    pad = e_pad - e
    src_pad = jnp.concatenate([src, jnp.zeros((pad,), jnp.int32)])
    dst_pad = jnp.concatenate([dst, jnp.full((pad,), n, jnp.int32)])

    deg_parts = _sc_degree(dst_pad, rows, rpw, nch, epw)
    d0 = deg_parts[:n].reshape(n, 1)
    d1 = deg_parts[rows:rows + n].reshape(n, 1)

    h1s, dinv = _tc_scale_matmul(x, W1, d0, d1, n, d_hid)

    parts = _sc_prop_wide(h1s, src_pad, dst_pad, rows, rpw, nch, epw, d_hid)
    p0 = parts[:n]
    p1 = parts[rows:rows + n]

    vs = _tc_layer2_in(p0, p1, h1s, dinv, b1.reshape(1, d_hid), W2, n)

    t_parts = _sc_prop_scalar(vs.reshape(n), src_pad, dst_pad, rows, rpw, nch, epw)
    t0 = t_parts[:n].reshape(n, 1)
    t1 = t_parts[rows:rows + n].reshape(n, 1)

    out = _tc_finish(t0, t1, vs, dinv, b2.reshape(1, 1), n)
    return out.reshape(n)


# speedup vs baseline: 19.9046x; 1.2655x over previous
import functools

import jax
import jax.numpy as jnp
from jax import lax
from jax.experimental import pallas as pl
from jax.experimental.pallas import tpu as pltpu
from jax.experimental.pallas import tpu_sc as plsc

NC = 2
NS = 16
CH = 64


def _mesh():
    return plsc.VectorSubcoreMesh(core_axis_name="c", subcore_axis_name="s")


def _sc_degree(dst_pad, rows, rpw, nch, epw):
    @functools.partial(
        pl.kernel,
        out_type=jax.ShapeDtypeStruct((NC * rows,), jnp.float32),
        mesh=_mesh(),
        scratch_types=[
            pltpu.VMEM((CH,), jnp.int32),
            pltpu.VMEM((CH,), jnp.int32),
            pltpu.VMEM((CH,), jnp.float32),
            pltpu.VMEM_SHARED((rows,), jnp.float32),
            pltpu.SemaphoreType.DMA,
            pltpu.SemaphoreType.DMA,
        ],
        name="sc_gcn_degree",
    )
    def deg_k(dst_hbm, out_hbm, db0, db1, vals, acc, semi0, semi1):
        cid = lax.axis_index("c")
        sid = lax.axis_index("s")
        base = (cid * NS + sid) * epw
        for k in range(CH // 16):
            vals[pl.ds(k * 16, 16)] = jnp.zeros((16,), jnp.float32)
        for r in range(rpw // CH):
            pltpu.sync_copy(vals, acc.at[pl.ds(sid * rpw + r * CH, CH)])
        plsc.subcore_barrier()
        for k in range(CH // 16):
            vals[pl.ds(k * 16, 16)] = jnp.ones((16,), jnp.float32)

        pltpu.async_copy(dst_hbm.at[pl.ds(base, CH)], db0, semi0)
        pltpu.async_copy(dst_hbm.at[pl.ds(base + CH, CH)], db1, semi1)

        def body(jj, carry):
            j = jj * 2
            o0 = base + jnp.minimum(j + 2, nch - 1) * CH
            o1 = base + jnp.minimum(j + 3, nch - 1) * CH
            pltpu.make_async_copy(dst_hbm.at[pl.ds(base, CH)], db0, semi0).wait()
            pltpu.sync_copy(vals, acc.at[db0], add=True)
            pltpu.async_copy(dst_hbm.at[pl.ds(o0, CH)], db0, semi0)
            pltpu.make_async_copy(dst_hbm.at[pl.ds(base, CH)], db1, semi1).wait()
            pltpu.sync_copy(vals, acc.at[db1], add=True)
            pltpu.async_copy(dst_hbm.at[pl.ds(o1, CH)], db1, semi1)
            return carry

        lax.fori_loop(0, nch // 2, body, 0)
        pltpu.make_async_copy(dst_hbm.at[pl.ds(base, CH)], db0, semi0).wait()
        pltpu.make_async_copy(dst_hbm.at[pl.ds(base, CH)], db1, semi1).wait()
        plsc.subcore_barrier()
        pltpu.sync_copy(acc.at[pl.ds(sid * rpw, rpw)],
                        out_hbm.at[pl.ds(cid * rows + sid * rpw, rpw)])

    return deg_k(dst_pad)


def _sc_prop_wide(table, src_pad, dst_pad, rows, rpw, nch, epw, d):
    @functools.partial(
        pl.kernel,
        out_type=jax.ShapeDtypeStruct((NC * rows, d), jnp.float32),
        mesh=_mesh(),
        scratch_types=[
            pltpu.VMEM((CH,), jnp.int32),
            pltpu.VMEM((CH,), jnp.int32),
            pltpu.VMEM((CH,), jnp.int32),
            pltpu.VMEM((CH,), jnp.int32),
            pltpu.VMEM((CH, d), jnp.float32),
            pltpu.VMEM((CH, d), jnp.float32),
            pltpu.VMEM_SHARED((rows, d), jnp.float32),
            pltpu.SemaphoreType.DMA,
            pltpu.SemaphoreType.DMA,
            pltpu.SemaphoreType.DMA,
            pltpu.SemaphoreType.DMA,
        ],
        name="sc_gcn_prop128",
    )
    def prop_k(tab_hbm, src_hbm, dst_hbm, out_hbm,
               sb0, sb1, db0, db1, g0, g1, acc,
               semi0, semi1, semg0, semg1):
        cid = lax.axis_index("c")
        sid = lax.axis_index("s")
        base = (cid * NS + sid) * epw

        def zbody(r, carry):
            for k in range(d // 16):
                g0[r, pl.ds(k * 16, 16)] = jnp.zeros((16,), jnp.float32)
            return carry

        lax.fori_loop(0, CH, zbody, 0)
        for r in range(rpw // CH):
            pltpu.sync_copy(g0, acc.at[pl.ds(sid * rpw + r * CH, CH)])
        plsc.subcore_barrier()

        pltpu.async_copy(src_hbm.at[pl.ds(base, CH)], sb0, semi0)
        pltpu.async_copy(dst_hbm.at[pl.ds(base, CH)], db0, semi0)
        pltpu.async_copy(src_hbm.at[pl.ds(base + CH, CH)], sb1, semi1)
        pltpu.async_copy(dst_hbm.at[pl.ds(base + CH, CH)], db1, semi1)
        pltpu.make_async_copy(src_hbm.at[pl.ds(base, CH)], sb0, semi0).wait()
        pltpu.make_async_copy(dst_hbm.at[pl.ds(base, CH)], db0, semi0).wait()
        pltpu.async_copy(tab_hbm.at[sb0], g0, semg0)

        def body(jj, carry):
            j = jj * 2
            o2 = base + jnp.minimum(j + 2, nch - 1) * CH
            o3 = base + jnp.minimum(j + 3, nch - 1) * CH
            pltpu.make_async_copy(src_hbm.at[pl.ds(base, CH)], sb1, semi1).wait()
            pltpu.make_async_copy(dst_hbm.at[pl.ds(base, CH)], db1, semi1).wait()
            pltpu.async_copy(tab_hbm.at[sb1], g1, semg1)
            pltpu.make_async_copy(tab_hbm.at[sb0], g0, semg0).wait()
            pltpu.sync_copy(g0, acc.at[db0], add=True)
            pltpu.async_copy(src_hbm.at[pl.ds(o2, CH)], sb0, semi0)
            pltpu.async_copy(dst_hbm.at[pl.ds(o2, CH)], db0, semi0)
            pltpu.make_async_copy(src_hbm.at[pl.ds(base, CH)], sb0, semi0).wait()
            pltpu.make_async_copy(dst_hbm.at[pl.ds(base, CH)], db0, semi0).wait()
            pltpu.async_copy(tab_hbm.at[sb0], g0, semg0)
            pltpu.make_async_copy(tab_hbm.at[sb1], g1, semg1).wait()
            pltpu.sync_copy(g1, acc.at[db1], add=True)
            pltpu.async_copy(src_hbm.at[pl.ds(o3, CH)], sb1, semi1)
            pltpu.async_copy(dst_hbm.at[pl.ds(o3, CH)], db1, semi1)
            return carry

        lax.fori_loop(0, nch // 2, body, 0)
        pltpu.make_async_copy(tab_hbm.at[sb0], g0, semg0).wait()
        pltpu.make_async_copy(src_hbm.at[pl.ds(base, CH)], sb1, semi1).wait()
        pltpu.make_async_copy(dst_hbm.at[pl.ds(base, CH)], db1, semi1).wait()
        plsc.subcore_barrier()
        pltpu.sync_copy(acc.at[pl.ds(sid * rpw, rpw)],
                        out_hbm.at[pl.ds(cid * rows + sid * rpw, rpw)])

    return prop_k(table, src_pad, dst_pad)


def _sc_prop_scalar(vec, src_pad, dst_pad, rows, rpw, nch, epw, n):
    @functools.partial(
        pl.kernel,
        out_type=jax.ShapeDtypeStruct((NC * rows,), jnp.float32),
        mesh=_mesh(),
        scratch_types=[
            pltpu.VMEM((CH,), jnp.int32),
            pltpu.VMEM((CH,), jnp.int32),
            pltpu.VMEM((CH,), jnp.int32),
            pltpu.VMEM((CH,), jnp.int32),
            pltpu.VMEM((CH,), jnp.float32),
            pltpu.VMEM((CH,), jnp.float32),
            pltpu.VMEM_SHARED((rows,), jnp.float32),
            pltpu.SemaphoreType.DMA,
            pltpu.SemaphoreType.DMA,
            pltpu.SemaphoreType.DMA,
            pltpu.SemaphoreType.DMA,
        ],
        name="sc_gcn_prop1",
    )
    def prop1_k(vec_hbm, src_hbm, dst_hbm, out_hbm,
                sb0, sb1, db0, db1, g0, g1, acc, semi0, semi1, semg0, semg1):
        cid = lax.axis_index("c")
        sid = lax.axis_index("s")
        base = (cid * NS + sid) * epw
        for k in range(CH // 16):
            g0[pl.ds(k * 16, 16)] = jnp.zeros((16,), jnp.float32)
        for r in range(rpw // CH):
            pltpu.sync_copy(g0, acc.at[pl.ds(sid * rpw + r * CH, CH)])
        plsc.subcore_barrier()

        pltpu.async_copy(src_hbm.at[pl.ds(base, CH)], sb0, semi0)
        pltpu.async_copy(dst_hbm.at[pl.ds(base, CH)], db0, semi0)
        pltpu.async_copy(src_hbm.at[pl.ds(base + CH, CH)], sb1, semi1)
        pltpu.async_copy(dst_hbm.at[pl.ds(base + CH, CH)], db1, semi1)
        pltpu.make_async_copy(src_hbm.at[pl.ds(base, CH)], sb0, semi0).wait()
        pltpu.make_async_copy(dst_hbm.at[pl.ds(base, CH)], db0, semi0).wait()
        pltpu.async_copy(vec_hbm.at[sb0], g0, semg0)

        def body(jj, carry):
            j = jj * 2
            o2 = base + jnp.minimum(j + 2, nch - 1) * CH
            o3 = base + jnp.minimum(j + 3, nch - 1) * CH
            pltpu.make_async_copy(src_hbm.at[pl.ds(base, CH)], sb1, semi1).wait()
            pltpu.make_async_copy(dst_hbm.at[pl.ds(base, CH)], db1, semi1).wait()
            pltpu.async_copy(vec_hbm.at[sb1], g1, semg1)
            pltpu.make_async_copy(vec_hbm.at[sb0], g0, semg0).wait()
            pltpu.sync_copy(g0, acc.at[db0], add=True)
            pltpu.async_copy(src_hbm.at[pl.ds(o2, CH)], sb0, semi0)
            pltpu.async_copy(dst_hbm.at[pl.ds(o2, CH)], db0, semi0)
            pltpu.make_async_copy(src_hbm.at[pl.ds(base, CH)], sb0, semi0).wait()
            pltpu.make_async_copy(dst_hbm.at[pl.ds(base, CH)], db0, semi0).wait()
            pltpu.async_copy(vec_hbm.at[sb0], g0, semg0)
            pltpu.make_async_copy(vec_hbm.at[sb1], g1, semg1).wait()
            pltpu.sync_copy(g1, acc.at[db1], add=True)
            pltpu.async_copy(src_hbm.at[pl.ds(o3, CH)], sb1, semi1)
            pltpu.async_copy(dst_hbm.at[pl.ds(o3, CH)], db1, semi1)
            return carry

        lax.fori_loop(0, nch // 2, body, 0)
        pltpu.make_async_copy(vec_hbm.at[sb0], g0, semg0).wait()
        pltpu.make_async_copy(src_hbm.at[pl.ds(base, CH)], sb1, semi1).wait()
        pltpu.make_async_copy(dst_hbm.at[pl.ds(base, CH)], db1, semi1).wait()
        plsc.subcore_barrier()
        pltpu.sync_copy(acc.at[pl.ds(sid * rpw, rpw)],
                        out_hbm.at[pl.ds(cid * rows + sid * rpw, rpw)])

    return prop1_k(vec, src_pad, dst_pad)


def _tc_scale_matmul(x, w1, d0, d1, n, d_hid):
    def body(x_ref, w_ref, d0_ref, d1_ref, h_ref, dinv_ref):
        deg = d0_ref[...] + d1_ref[...] + 1.0
        dinv = lax.rsqrt(jnp.maximum(deg, 1.0))
        h = jnp.dot(x_ref[...], w_ref[...], preferred_element_type=jnp.float32)
        h_ref[...] = h * dinv
        dinv_ref[...] = dinv

    return pl.pallas_call(
        body,
        out_shape=(jax.ShapeDtypeStruct((n, d_hid), jnp.float32),
                   jax.ShapeDtypeStruct((n, 1), jnp.float32)),
    )(x, w1, d0, d1)


def _tc_layer2_in(p0, p1, h1s, dinv, b1, w2, n):
    def body(p0_ref, p1_ref, h_ref, dinv_ref, b1_ref, w2_ref, vs_ref):
        out1 = dinv_ref[...] * (p0_ref[...] + p1_ref[...] + h_ref[...]) + b1_ref[...]
        hrelu = jnp.maximum(out1, 0.0)
        v = jnp.dot(hrelu, w2_ref[...], preferred_element_type=jnp.float32)
        vs_ref[...] = dinv_ref[...] * v

    return pl.pallas_call(
        body,
        out_shape=jax.ShapeDtypeStruct((n, 1), jnp.float32),
    )(p0, p1, h1s, dinv, b1, w2)


def _tc_finish(t0, t1, vs, dinv, b2, n):
    def body(t0_ref, t1_ref, vs_ref, dinv_ref, b2_ref, o_ref):
        z = dinv_ref[...] * (t0_ref[...] + t1_ref[...] + vs_ref[...]) + b2_ref[...]
        o_ref[...] = 1.0 / (1.0 + jnp.exp(-z))

    return pl.pallas_call(
        body,
        out_shape=jax.ShapeDtypeStruct((n, 1), jnp.float32),
    )(t0, t1, vs, dinv, b2)


def kernel(x, edge_index, W1, b1, W2, b2):
    n, d_in = x.shape
    d_hid = W1.shape[1]
    e = edge_index.shape[1]

    nw = NC * NS
    nch = -(-e // (nw * CH))
    nch += nch % 2
    epw = nch * CH
    e_pad = epw * nw
    rpw = -(-(n + 1) // (NS * CH)) * CH
    rows = rpw * NS

    src = edge_index[0]
    dst = edge_index[1]
    pad = e_pad - e
    src_pad = jnp.concatenate([src, jnp.zeros((pad,), jnp.int32)])
    dst_pad = jnp.concatenate([dst, jnp.full((pad,), n, jnp.int32)])

    deg_parts = _sc_degree(dst_pad, rows, rpw, nch, epw)
    d0 = deg_parts[:n].reshape(n, 1)
    d1 = deg_parts[rows:rows + n].reshape(n, 1)

    h1s, dinv = _tc_scale_matmul(x, W1, d0, d1, n, d_hid)

    parts = _sc_prop_wide(h1s, src_pad, dst_pad, rows, rpw, nch, epw, d_hid)
    p0 = parts[:n]
    p1 = parts[rows:rows + n]

    vs = _tc_layer2_in(p0, p1, h1s, dinv, b1.reshape(1, d_hid), W2, n)

    t_parts = _sc_prop_scalar(vs.reshape(n), src_pad, dst_pad, rows, rpw, nch, epw, n)
    t0 = t_parts[:n].reshape(n, 1)
    t1 = t_parts[rows:rows + n].reshape(n, 1)

    out = _tc_finish(t0, t1, vs, dinv, b2.reshape(1, 1), n)
    return out.reshape(n)
